# baseline (speedup 1.0000x reference)
import jax
import jax.numpy as jnp
from jax.experimental import pallas as pl

_B, _S, _D = 4096, 200, 64
_ROW = _S * _D
_BLK = 128


def _add_body(x_ref, pos_ref, o_ref):
    o_ref[...] = x_ref[...] + pos_ref[...]


def kernel(x, pos_emb):
    x2 = x.reshape(_B, _ROW)
    pos2 = pos_emb.reshape(1, _ROW)
    out = pl.pallas_call(
        _add_body,
        grid=(_B // _BLK,),
        in_specs=[
            pl.BlockSpec((_BLK, _ROW), lambda i: (i, 0)),
            pl.BlockSpec((1, _ROW), lambda i: (0, 0)),
        ],
        out_specs=pl.BlockSpec((_BLK, _ROW), lambda i: (i, 0)),
        out_shape=jax.ShapeDtypeStruct((_B, _ROW), jnp.float32),
    )(x2, pos2)
    return out.reshape(_B, _S, _D)
